# baseline (speedup 1.0000x reference)
import jax
import jax.numpy as jnp
from jax.experimental import pallas as pl
from jax.experimental.pallas import tpu as pltpu

_NB = 512
_TAPS = ((0, 0), (0, 1), (1, 0), (1, 1))


def _lenet_kernel(x_ref, w1_ref, b1_ref, w2_ref, b2_ref,
                  fw1_ref, fb1_ref, fw2_ref, fb2_ref, fw3_ref, fb3_ref,
                  o_ref):
    x5 = x_ref[...].reshape(3, 32, 8, 4, _NB)
    xp = [x5[:, :, :, s, :] for s in range(4)]

    cols = []
    for o in range(8):
        for u in range(8):
            s, j0 = u % 4, u // 4
            rows = [xp[s][:, 4 * i + o, j0:j0 + 7, :] for i in range(7)]
            cols.append(jnp.stack(rows, axis=1))
    p1 = jnp.stack(cols, axis=0).reshape(192, 7, 7, _NB)

    t1 = jax.lax.dot_general(
        w1_ref[...], p1, (((1,), (0,)), ((), ())),
        preferred_element_type=jnp.float32).reshape(4, 4, 6, 7, 7, _NB)

    b1 = b1_ref[...].reshape(6, 1, 1, 1)
    b2 = b2_ref[...].reshape(16, 1, 1, 1)

    y1 = [[None, None], [None, None]]
    for p in (0, 1):
        for q in (0, 1):
            pq = t1[2 * p + q]
            z = jnp.maximum(jnp.maximum(pq[0], pq[1]),
                            jnp.maximum(pq[2], pq[3]))
            y1[p][q] = jnp.maximum(z + b1, 0.0)

    cols = []
    for a in range(6):
        for b in range(6):
            cols.append(y1[a % 2][b % 2][:, a // 2:a // 2 + 5,
                                         b // 2:b // 2 + 5, :])
    p2 = jnp.stack(cols, axis=0).reshape(216, 5, 5, _NB)

    t2 = jax.lax.dot_general(
        w2_ref[...], p2, (((1,), (0,)), ((), ())),
        preferred_element_type=jnp.float32).reshape(4, 16, 5, 5, _NB)
    z = jnp.maximum(jnp.maximum(t2[0], t2[1]), jnp.maximum(t2[2], t2[3]))
    y2 = jnp.maximum(z + b2, 0.0)

    xf = y2.reshape(400, _NB)
    h = jnp.dot(fw1_ref[...].astype(jnp.float32), xf,
                preferred_element_type=jnp.float32)
    h = jnp.maximum(h + fb1_ref[...], 0.0)
    h = jnp.dot(fw2_ref[...].astype(jnp.float32), h,
                preferred_element_type=jnp.float32)
    h = jnp.maximum(h + fb2_ref[...], 0.0)
    h = jnp.dot(fw3_ref[...].astype(jnp.float32), h,
                preferred_element_type=jnp.float32)
    o_ref[...] = h + fb3_ref[...]


def _stack_conv1_weights(cw1):
    w = cw1.reshape(6, 3, 5, 5).transpose(0, 2, 3, 1).astype(jnp.float32)
    rows = []
    for p in (0, 1):
        for q in (0, 1):
            for dh, dw in _TAPS:
                a, b = 2 * p + dh, 2 * q + dw
                rows.append(jnp.pad(
                    w, ((0, 0), (a, 3 - a), (b, 3 - b), (0, 0))))
    return jnp.stack(rows).reshape(96, 192)


def _stack_conv2_weights(cw2):
    w = cw2.reshape(16, 6, 5, 5).transpose(0, 2, 3, 1).astype(jnp.float32)
    rows = [jnp.pad(w, ((0, 0), (dh, 1 - dh), (dw, 1 - dw), (0, 0)))
            for dh, dw in _TAPS]
    return jnp.stack(rows).reshape(64, 216)


def kernel(x, cw1, cb1, cw2, cb2, ftw1, ftb1, ftw2, ftb2, ftw3, ftb3):
    N = x.shape[0]
    x3 = x.transpose(1, 2, 3, 0)
    w1 = _stack_conv1_weights(cw1)
    w2 = _stack_conv2_weights(cw2)

    out = pl.pallas_call(
        _lenet_kernel,
        out_shape=jax.ShapeDtypeStruct((16, N), jnp.float32),
        grid=(N // _NB,),
        in_specs=[
            pl.BlockSpec((3, 32, 32, _NB), lambda i: (0, 0, 0, i)),
            pl.BlockSpec((96, 192), lambda i: (0, 0)),
            pl.BlockSpec((6, 1), lambda i: (0, 0)),
            pl.BlockSpec((64, 216), lambda i: (0, 0)),
            pl.BlockSpec((16, 1), lambda i: (0, 0)),
            pl.BlockSpec((128, 400), lambda i: (0, 0)),
            pl.BlockSpec((128, 1), lambda i: (0, 0)),
            pl.BlockSpec((128, 128), lambda i: (0, 0)),
            pl.BlockSpec((128, 1), lambda i: (0, 0)),
            pl.BlockSpec((16, 128), lambda i: (0, 0)),
            pl.BlockSpec((16, 1), lambda i: (0, 0)),
        ],
        out_specs=pl.BlockSpec((16, _NB), lambda i: (0, i)),
        compiler_params=pltpu.CompilerParams(
            dimension_semantics=("parallel",),
            vmem_limit_bytes=60 * 1024 * 1024),
    )(x3, w1, cb1, w2, cb2, ftw1, ftb1, ftw2, ftb2, ftw3, ftb3)
    return out[:10, :].T


# speedup vs baseline: 1.0604x; 1.0604x over previous
import jax
import jax.numpy as jnp
from jax.experimental import pallas as pl
from jax.experimental.pallas import tpu as pltpu

_NB = 256
_TAPS = ((0, 0), (0, 1), (1, 0), (1, 1))


def _lenet_kernel(x_ref, w1_ref, b1_ref, w2_ref, b2_ref,
                  fw1_ref, fb1_ref, fw2_ref, fb2_ref, fw3_ref, fb3_ref,
                  o_ref):
    x5 = x_ref[...].reshape(3, 32, 8, 4, _NB)
    xp = [x5[:, :, :, s, :] for s in range(4)]

    cols = []
    for o in range(8):
        for u in range(8):
            s, j0 = u % 4, u // 4
            rows = [xp[s][:, 4 * i + o, j0:j0 + 7, :] for i in range(7)]
            cols.append(jnp.stack(rows, axis=1))
    p1 = jnp.stack(cols, axis=0).reshape(192, 7, 7, _NB)

    t1 = jax.lax.dot_general(
        w1_ref[...], p1, (((1,), (0,)), ((), ())),
        preferred_element_type=jnp.float32).reshape(4, 4, 6, 7, 7, _NB)

    b1 = b1_ref[...].reshape(6, 1, 1, 1)
    b2 = b2_ref[...].reshape(16, 1, 1, 1)

    y1 = [[None, None], [None, None]]
    for p in (0, 1):
        for q in (0, 1):
            pq = t1[2 * p + q]
            z = jnp.maximum(jnp.maximum(pq[0], pq[1]),
                            jnp.maximum(pq[2], pq[3]))
            y1[p][q] = jnp.maximum(z + b1, 0.0)

    cols = []
    for a in range(6):
        for b in range(6):
            cols.append(y1[a % 2][b % 2][:, a // 2:a // 2 + 5,
                                         b // 2:b // 2 + 5, :])
    p2 = jnp.stack(cols, axis=0).reshape(216, 5, 5, _NB)

    t2 = jax.lax.dot_general(
        w2_ref[...], p2, (((1,), (0,)), ((), ())),
        preferred_element_type=jnp.float32).reshape(4, 16, 5, 5, _NB)
    z = jnp.maximum(jnp.maximum(t2[0], t2[1]), jnp.maximum(t2[2], t2[3]))
    y2 = jnp.maximum(z + b2, 0.0)

    xf = y2.reshape(400, _NB)
    h = jnp.dot(fw1_ref[...].astype(jnp.float32), xf,
                preferred_element_type=jnp.float32)
    h = jnp.maximum(h + fb1_ref[...], 0.0)
    h = jnp.dot(fw2_ref[...].astype(jnp.float32), h,
                preferred_element_type=jnp.float32)
    h = jnp.maximum(h + fb2_ref[...], 0.0)
    h = jnp.dot(fw3_ref[...].astype(jnp.float32), h,
                preferred_element_type=jnp.float32)
    o_ref[...] = h + fb3_ref[...]


def _stack_conv1_weights(cw1):
    w = cw1.reshape(6, 3, 5, 5).transpose(0, 2, 3, 1).astype(jnp.float32)
    rows = []
    for p in (0, 1):
        for q in (0, 1):
            for dh, dw in _TAPS:
                a, b = 2 * p + dh, 2 * q + dw
                rows.append(jnp.pad(
                    w, ((0, 0), (a, 3 - a), (b, 3 - b), (0, 0))))
    return jnp.stack(rows).reshape(96, 192)


def _stack_conv2_weights(cw2):
    w = cw2.reshape(16, 6, 5, 5).transpose(0, 2, 3, 1).astype(jnp.float32)
    rows = [jnp.pad(w, ((0, 0), (dh, 1 - dh), (dw, 1 - dw), (0, 0)))
            for dh, dw in _TAPS]
    return jnp.stack(rows).reshape(64, 216)


def kernel(x, cw1, cb1, cw2, cb2, ftw1, ftb1, ftw2, ftb2, ftw3, ftb3):
    N = x.shape[0]
    x3 = x.transpose(1, 2, 3, 0)
    w1 = _stack_conv1_weights(cw1)
    w2 = _stack_conv2_weights(cw2)

    out = pl.pallas_call(
        _lenet_kernel,
        out_shape=jax.ShapeDtypeStruct((16, N), jnp.float32),
        grid=(N // _NB,),
        in_specs=[
            pl.BlockSpec((3, 32, 32, _NB), lambda i: (0, 0, 0, i)),
            pl.BlockSpec((96, 192), lambda i: (0, 0)),
            pl.BlockSpec((6, 1), lambda i: (0, 0)),
            pl.BlockSpec((64, 216), lambda i: (0, 0)),
            pl.BlockSpec((16, 1), lambda i: (0, 0)),
            pl.BlockSpec((128, 400), lambda i: (0, 0)),
            pl.BlockSpec((128, 1), lambda i: (0, 0)),
            pl.BlockSpec((128, 128), lambda i: (0, 0)),
            pl.BlockSpec((128, 1), lambda i: (0, 0)),
            pl.BlockSpec((16, 128), lambda i: (0, 0)),
            pl.BlockSpec((16, 1), lambda i: (0, 0)),
        ],
        out_specs=pl.BlockSpec((16, _NB), lambda i: (0, i)),
        compiler_params=pltpu.CompilerParams(
            dimension_semantics=("parallel",),
            vmem_limit_bytes=48 * 1024 * 1024),
    )(x3, w1, cb1, w2, cb2, ftw1, ftb1, ftw2, ftb2, ftw3, ftb3)
    return out[:10, :].T
